# baseline (speedup 1.0000x reference)
import functools

import jax
import jax.numpy as jnp
from jax import lax
from jax.experimental import pallas as pl
from jax.experimental.pallas import tpu as pltpu

_EPS = 1e-5


def _silu(v):
    return v * jax.nn.sigmoid(v)


def _stats_body(n_rows, x_ref, g_ref, b_ref, ab_ref, acc_ref):
    i = pl.program_id(0)
    nb = pl.num_programs(0)
    blk = x_ref[...]

    @pl.when(i == 0)
    def _():
        acc_ref[...] = jnp.zeros_like(acc_ref)

    acc_ref[0:1, :] += jnp.sum(blk, axis=0, keepdims=True)
    acc_ref[1:2, :] += jnp.sum(blk * blk, axis=0, keepdims=True)

    @pl.when(i == nb - 1)
    def _():
        mu = acc_ref[0:1, :] / n_rows
        var = acc_ref[1:2, :] / n_rows - mu * mu
        a = g_ref[...] * lax.rsqrt(var + _EPS)
        ab_ref[0:1, :] = a
        ab_ref[1:2, :] = b_ref[...] - mu * a


def _bn_stats(x, g, b, bn):
    n, f = x.shape
    grid = n // bn
    return pl.pallas_call(
        functools.partial(_stats_body, float(n)),
        grid=(grid,),
        in_specs=[
            pl.BlockSpec((bn, f), lambda i: (i, 0)),
            pl.BlockSpec((1, f), lambda i: (0, 0)),
            pl.BlockSpec((1, f), lambda i: (0, 0)),
        ],
        out_specs=pl.BlockSpec((2, f), lambda i: (0, 0)),
        out_shape=jax.ShapeDtypeStruct((2, f), jnp.float32),
        scratch_shapes=[pltpu.VMEM((2, f), jnp.float32)],
    )(x, g.reshape(1, f), b.reshape(1, f))


def _conv_tab_body(kvol, x_ref, ab_ref, w_ref, o_ref):
    a = ab_ref[0:1, :]
    b = ab_ref[1:2, :]
    h = x_ref[...] * a + b
    h = _silu(h)
    for k in range(kvol):
        o_ref[k] = jnp.dot(h, w_ref[k], preferred_element_type=jnp.float32)


def _conv_table(x, ab, w, bn):
    n, f = x.shape
    kvol, _, fo = w.shape
    grid = n // bn
    return pl.pallas_call(
        functools.partial(_conv_tab_body, kvol),
        grid=(grid,),
        in_specs=[
            pl.BlockSpec((bn, f), lambda i: (i, 0)),
            pl.BlockSpec((2, f), lambda i: (0, 0)),
            pl.BlockSpec((kvol, f, fo), lambda i: (0, 0, 0)),
        ],
        out_specs=pl.BlockSpec((kvol, bn, fo), lambda i: (0, i, 0)),
        out_shape=jax.ShapeDtypeStruct((kvol, n, fo), jnp.float32),
    )(x, ab, w)


def _epi1_body(n_rows, nbatch, acc_ref, b1_ref, t_ref, wt_ref, bt_ref,
               bi_ref, h_ref, ab_ref, sacc_ref, temb_ref):
    i = pl.program_id(0)
    nb = pl.num_programs(0)

    @pl.when(i == 0)
    def _():
        temb_ref[...] = (
            jnp.dot(_silu(t_ref[...]), wt_ref[...],
                    preferred_element_type=jnp.float32) + bt_ref[...])
        sacc_ref[...] = jnp.zeros_like(sacc_ref)

    bi = bi_ref[0, 0, :]
    oh = (bi[:, None] == lax.broadcasted_iota(jnp.int32, (bi.shape[0], nbatch), 1)
          ).astype(jnp.float32)
    tembp = jnp.dot(oh, temb_ref[0:nbatch, :], preferred_element_type=jnp.float32)
    h = acc_ref[...] + b1_ref[...] + tembp
    h_ref[...] = h

    sacc_ref[0:1, :] += jnp.sum(h, axis=0, keepdims=True)
    sacc_ref[1:2, :] += jnp.sum(h * h, axis=0, keepdims=True)

    @pl.when(i == nb - 1)
    def _():
        mu = sacc_ref[0:1, :] / n_rows
        var = sacc_ref[1:2, :] / n_rows - mu * mu
        ab_ref[0:1, :] = lax.rsqrt(var + _EPS)
        ab_ref[1:2, :] = -mu * lax.rsqrt(var + _EPS)


def _epilogue1(acc, b1, t, wt, bt, bidx, g2, b2g, bn):
    n, f = acc.shape
    nbatch, temb_dim = t.shape
    tpad = jnp.zeros((8, temb_dim), jnp.float32).at[0:nbatch].set(t)
    bidx3 = bidx.reshape(n // bn, 1, bn)
    grid = n // bn
    h, ab = pl.pallas_call(
        functools.partial(_epi1_body, float(n), nbatch),
        grid=(grid,),
        in_specs=[
            pl.BlockSpec((bn, f), lambda i: (i, 0)),
            pl.BlockSpec((1, f), lambda i: (0, 0)),
            pl.BlockSpec((8, temb_dim), lambda i: (0, 0)),
            pl.BlockSpec((temb_dim, f), lambda i: (0, 0)),
            pl.BlockSpec((1, f), lambda i: (0, 0)),
            pl.BlockSpec((1, 1, bn), lambda i: (i, 0, 0)),
        ],
        out_specs=[
            pl.BlockSpec((bn, f), lambda i: (i, 0)),
            pl.BlockSpec((2, f), lambda i: (0, 0)),
        ],
        out_shape=[
            jax.ShapeDtypeStruct((n, f), jnp.float32),
            jax.ShapeDtypeStruct((2, f), jnp.float32),
        ],
        scratch_shapes=[
            pltpu.VMEM((2, f), jnp.float32),
            pltpu.VMEM((8, f), jnp.float32),
        ],
    )(acc, b1.reshape(1, f), tpad, wt, bt.reshape(1, f), bidx3)
    a = ab[0:1] * g2.reshape(1, f)
    b = ab[1:2] * g2.reshape(1, f) + b2g.reshape(1, f)
    return h, jnp.concatenate([a, b], axis=0)


def _final_body(acc_ref, b2_ref, x_ref, o_ref):
    o_ref[...] = acc_ref[...] + b2_ref[...] + x_ref[...]


def _final(acc, b2, x, bn):
    n, f = x.shape
    grid = n // bn
    return pl.pallas_call(
        _final_body,
        grid=(grid,),
        in_specs=[
            pl.BlockSpec((bn, f), lambda i: (i, 0)),
            pl.BlockSpec((1, f), lambda i: (0, 0)),
            pl.BlockSpec((bn, f), lambda i: (i, 0)),
        ],
        out_specs=pl.BlockSpec((bn, f), lambda i: (i, 0)),
        out_shape=jax.ShapeDtypeStruct((n, f), jnp.float32),
    )(acc, b2.reshape(1, f), x)


def kernel(x, t, edge_index, kernel_offset, batch_idx,
           bn1_g, bn1_b, W1, b1, bn2_g, bn2_b, W2, b2, Wt, bt):
    n, f = x.shape
    kvol = W1.shape[0]
    bn = 200 if n % 200 == 0 else 8

    src = edge_index[0].astype(jnp.int32)
    dst = edge_index[1].astype(jnp.int32)
    koff = kernel_offset.astype(jnp.int32)
    gidx = koff * n + src
    bidx = batch_idx.astype(jnp.int32)

    ab1 = _bn_stats(x, bn1_g, bn1_b, bn)
    tab1 = _conv_table(x, ab1, W1, bn)
    msg = tab1.reshape(kvol * n, f)[gidx]
    acc1 = jnp.zeros((n, f), jnp.float32).at[dst].add(msg)

    h1, ab2 = _epilogue1(acc1, b1, t, Wt, bt, bidx, bn2_g, bn2_b, bn)

    tab2 = _conv_table(h1, ab2, W2, bn)
    msg2 = tab2.reshape(kvol * n, f)[gidx]
    acc2 = jnp.zeros((n, f), jnp.float32).at[dst].add(msg2)

    return _final(acc2, b2, x, bn)


# speedup vs baseline: 1.8899x; 1.8899x over previous
import functools

import jax
import jax.numpy as jnp
from jax import lax
from jax.experimental import pallas as pl
from jax.experimental.pallas import tpu as pltpu
from jax.experimental.pallas import tpu_sc as plsc

_EPS = 1e-5
_CH = 128
_NSUB = 16


def _silu(v):
    return v * jax.nn.sigmoid(v)


def _stats_body(n_rows, x_ref, g_ref, b_ref, ab_ref, acc_ref):
    i = pl.program_id(0)
    nb = pl.num_programs(0)
    blk = x_ref[...]

    @pl.when(i == 0)
    def _():
        acc_ref[...] = jnp.zeros_like(acc_ref)

    acc_ref[0:1, :] += jnp.sum(blk, axis=0, keepdims=True)
    acc_ref[1:2, :] += jnp.sum(blk * blk, axis=0, keepdims=True)

    @pl.when(i == nb - 1)
    def _():
        mu = acc_ref[0:1, :] / n_rows
        var = acc_ref[1:2, :] / n_rows - mu * mu
        a = g_ref[...] * lax.rsqrt(var + _EPS)
        ab_ref[0:1, :] = a
        ab_ref[1:2, :] = b_ref[...] - mu * a


def _bn_stats(x, g, b, bn):
    n, f = x.shape
    grid = n // bn
    return pl.pallas_call(
        functools.partial(_stats_body, float(n)),
        grid=(grid,),
        in_specs=[
            pl.BlockSpec((bn, f), lambda i: (i, 0)),
            pl.BlockSpec((1, f), lambda i: (0, 0)),
            pl.BlockSpec((1, f), lambda i: (0, 0)),
        ],
        out_specs=pl.BlockSpec((2, f), lambda i: (0, 0)),
        out_shape=jax.ShapeDtypeStruct((2, f), jnp.float32),
        scratch_shapes=[pltpu.VMEM((2, f), jnp.float32)],
    )(x, g.reshape(1, f), b.reshape(1, f))


def _conv_tab_body(kvol, x_ref, ab_ref, w_ref, o_ref):
    a = ab_ref[0:1, :]
    b = ab_ref[1:2, :]
    h = x_ref[...] * a + b
    h = _silu(h)
    for k in range(kvol):
        m = jnp.dot(h, w_ref[k], preferred_element_type=jnp.float32)
        o_ref[0, k] = m[:, :128]
        o_ref[1, k] = m[:, 128:]


def _conv_table(x, ab, w, bn):
    n, f = x.shape
    kvol, _, fo = w.shape
    fh = fo // 2
    grid = n // bn
    tab = pl.pallas_call(
        functools.partial(_conv_tab_body, kvol),
        grid=(grid,),
        in_specs=[
            pl.BlockSpec((bn, f), lambda i: (i, 0)),
            pl.BlockSpec((2, f), lambda i: (0, 0)),
            pl.BlockSpec((kvol, f, fo), lambda i: (0, 0, 0)),
        ],
        out_specs=pl.BlockSpec((2, kvol, bn, fh), lambda i: (0, 0, i, 0)),
        out_shape=jax.ShapeDtypeStruct((2, kvol, n, fh), jnp.float32),
    )(x, ab, w)
    return tab.reshape(2 * kvol * n, fh)


def _sc_gs_body(ep16, nacc, n_out, fh,
                tab, gidx2, dst, zeros,
                out, idx_v, dst_v, rows_v, acc, sem):
    c = lax.axis_index("c")
    s = lax.axis_index("s")

    rz = nacc // _NSUB
    pltpu.sync_copy(zeros.at[pl.ds(s * rz, rz)], acc.at[pl.ds(s * rz, rz)])
    plsc.subcore_barrier()

    nch = ep16 // _CH

    def body(j, carry):
        base = s * ep16 + j * _CH
        pltpu.sync_copy(gidx2.at[c, pl.ds(base, _CH)], idx_v)
        pltpu.sync_copy(dst.at[pl.ds(base, _CH)], dst_v)
        pltpu.async_copy(tab.at[idx_v], rows_v, sem).wait()
        pltpu.sync_copy(rows_v, acc.at[dst_v], add=True)
        return carry

    lax.fori_loop(0, nch, body, 0)
    plsc.subcore_barrier()

    ro = n_out // _NSUB
    pltpu.sync_copy(acc.at[pl.ds(s * ro, ro)],
                    out.at[c, pl.ds(s * ro, ro)])


def _sc_gather_scatter(tab, gidx2, dst_pad, zeros, n_out):
    fh = tab.shape[1]
    ep16 = dst_pad.shape[0] // _NSUB
    nacc = zeros.shape[0]
    mesh = plsc.VectorSubcoreMesh(core_axis_name="c", subcore_axis_name="s")
    body = functools.partial(_sc_gs_body, ep16, nacc, n_out, fh)
    out = pl.kernel(
        body,
        out_type=jax.ShapeDtypeStruct((2, n_out, fh), jnp.float32),
        mesh=mesh,
        scratch_types=[
            pltpu.VMEM((_CH,), jnp.int32),
            pltpu.VMEM((_CH,), jnp.int32),
            pltpu.VMEM((_CH, fh), jnp.float32),
            pltpu.VMEM_SHARED((nacc, fh), jnp.float32),
            pltpu.SemaphoreType.DMA,
        ],
    )(tab, gidx2, dst_pad, zeros)
    return out[0], out[1]


def _epi1_body(n_rows, nbatch, acca_ref, accb_ref, b1_ref, t_ref, wt_ref,
               bt_ref, bi_ref, h_ref, ab_ref, sacc_ref, temb_ref):
    i = pl.program_id(0)
    nb = pl.num_programs(0)

    @pl.when(i == 0)
    def _():
        temb_ref[...] = (
            jnp.dot(_silu(t_ref[...]), wt_ref[...],
                    preferred_element_type=jnp.float32) + bt_ref[...])
        sacc_ref[...] = jnp.zeros_like(sacc_ref)

    bi = bi_ref[0, 0, :]
    oh = (bi[:, None] == lax.broadcasted_iota(jnp.int32, (bi.shape[0], nbatch), 1)
          ).astype(jnp.float32)
    tembp = jnp.dot(oh, temb_ref[0:nbatch, :], preferred_element_type=jnp.float32)
    h = (jnp.concatenate([acca_ref[...], accb_ref[...]], axis=1)
         + b1_ref[...] + tembp)
    h_ref[...] = h

    sacc_ref[0:1, :] += jnp.sum(h, axis=0, keepdims=True)
    sacc_ref[1:2, :] += jnp.sum(h * h, axis=0, keepdims=True)

    @pl.when(i == nb - 1)
    def _():
        mu = sacc_ref[0:1, :] / n_rows
        var = sacc_ref[1:2, :] / n_rows - mu * mu
        r = lax.rsqrt(var + _EPS)
        ab_ref[0:1, :] = r
        ab_ref[1:2, :] = -mu * r


def _epilogue1(acca, accb, b1, t, wt, bt, bidx, g2, b2g, bn, n):
    fh = acca.shape[1]
    f = 2 * fh
    nbatch, temb_dim = t.shape
    tpad = jnp.zeros((8, temb_dim), jnp.float32).at[0:nbatch].set(t)
    bidx3 = bidx.reshape(n // bn, 1, bn)
    grid = n // bn
    h, ab = pl.pallas_call(
        functools.partial(_epi1_body, float(n), nbatch),
        grid=(grid,),
        in_specs=[
            pl.BlockSpec((bn, fh), lambda i: (i, 0)),
            pl.BlockSpec((bn, fh), lambda i: (i, 0)),
            pl.BlockSpec((1, f), lambda i: (0, 0)),
            pl.BlockSpec((8, temb_dim), lambda i: (0, 0)),
            pl.BlockSpec((temb_dim, f), lambda i: (0, 0)),
            pl.BlockSpec((1, f), lambda i: (0, 0)),
            pl.BlockSpec((1, 1, bn), lambda i: (i, 0, 0)),
        ],
        out_specs=[
            pl.BlockSpec((bn, f), lambda i: (i, 0)),
            pl.BlockSpec((2, f), lambda i: (0, 0)),
        ],
        out_shape=[
            jax.ShapeDtypeStruct((n, f), jnp.float32),
            jax.ShapeDtypeStruct((2, f), jnp.float32),
        ],
        scratch_shapes=[
            pltpu.VMEM((2, f), jnp.float32),
            pltpu.VMEM((8, f), jnp.float32),
        ],
    )(acca, accb, b1.reshape(1, f), tpad, wt, bt.reshape(1, f), bidx3)
    a = ab[0:1] * g2.reshape(1, f)
    b = ab[1:2] * g2.reshape(1, f) + b2g.reshape(1, f)
    return h, jnp.concatenate([a, b], axis=0)


def _final_body(acca_ref, accb_ref, b2_ref, x_ref, o_ref):
    o_ref[...] = (jnp.concatenate([acca_ref[...], accb_ref[...]], axis=1)
                  + b2_ref[...] + x_ref[...])


def _final(acca, accb, b2, x, bn):
    n, f = x.shape
    fh = acca.shape[1]
    grid = n // bn
    return pl.pallas_call(
        _final_body,
        grid=(grid,),
        in_specs=[
            pl.BlockSpec((bn, fh), lambda i: (i, 0)),
            pl.BlockSpec((bn, fh), lambda i: (i, 0)),
            pl.BlockSpec((1, f), lambda i: (0, 0)),
            pl.BlockSpec((bn, f), lambda i: (i, 0)),
        ],
        out_specs=pl.BlockSpec((bn, f), lambda i: (i, 0)),
        out_shape=jax.ShapeDtypeStruct((n, f), jnp.float32),
    )(acca, accb, b2.reshape(1, f), x)


def kernel(x, t, edge_index, kernel_offset, batch_idx,
           bn1_g, bn1_b, W1, b1, bn2_g, bn2_b, W2, b2, Wt, bt):
    n, f = x.shape
    e = edge_index.shape[1]
    kvol = W1.shape[0]
    bn = 200 if n % 200 == 0 else 8

    src = edge_index[0].astype(jnp.int32)
    dst = edge_index[1].astype(jnp.int32)
    koff = kernel_offset.astype(jnp.int32)
    gidx = koff * n + src
    bidx = batch_idx.astype(jnp.int32)

    step = _NSUB * _CH
    e_pad = ((e + step - 1) // step) * step
    n_out = ((n + 127) // 128) * 128
    nacc = n_out + 128
    gidx_pad = jnp.zeros((e_pad,), jnp.int32).at[:e].set(gidx)
    gidx2 = jnp.stack([gidx_pad, gidx_pad + kvol * n])
    dst_pad = jnp.full((e_pad,), n_out, jnp.int32).at[:e].set(dst)
    zeros = jnp.zeros((nacc, f // 2), jnp.float32)

    ab1 = _bn_stats(x, bn1_g, bn1_b, bn)
    tab1 = _conv_table(x, ab1, W1, bn)
    acc1a, acc1b = _sc_gather_scatter(tab1, gidx2, dst_pad, zeros, n_out)

    h1, ab2 = _epilogue1(acc1a, acc1b, b1, t, Wt, bt, bidx, bn2_g, bn2_b, bn, n)

    tab2 = _conv_table(h1, ab2, W2, bn)
    acc2a, acc2b = _sc_gather_scatter(tab2, gidx2, dst_pad, zeros, n_out)

    return _final(acc2a, acc2b, b2, x, bn)


# speedup vs baseline: 1.9138x; 1.0126x over previous
import functools

import jax
import jax.numpy as jnp
from jax import lax
from jax.experimental import pallas as pl
from jax.experimental.pallas import tpu as pltpu
from jax.experimental.pallas import tpu_sc as plsc

_EPS = 1e-5
_CH = 128
_NSUB = 16
_GRP = 16


def _silu(v):
    return v * jax.nn.sigmoid(v)


def _stats_body(n_rows, x_ref, g_ref, b_ref, ab_ref, acc_ref):
    i = pl.program_id(0)
    nb = pl.num_programs(0)
    blk = x_ref[...]

    @pl.when(i == 0)
    def _():
        acc_ref[...] = jnp.zeros_like(acc_ref)

    acc_ref[0:1, :] += jnp.sum(blk, axis=0, keepdims=True)
    acc_ref[1:2, :] += jnp.sum(blk * blk, axis=0, keepdims=True)

    @pl.when(i == nb - 1)
    def _():
        mu = acc_ref[0:1, :] / n_rows
        var = acc_ref[1:2, :] / n_rows - mu * mu
        a = g_ref[...] * lax.rsqrt(var + _EPS)
        ab_ref[0:1, :] = a
        ab_ref[1:2, :] = b_ref[...] - mu * a


def _bn_stats(x, g, b, bn):
    n, f = x.shape
    grid = n // bn
    return pl.pallas_call(
        functools.partial(_stats_body, float(n)),
        grid=(grid,),
        in_specs=[
            pl.BlockSpec((bn, f), lambda i: (i, 0)),
            pl.BlockSpec((1, f), lambda i: (0, 0)),
            pl.BlockSpec((1, f), lambda i: (0, 0)),
        ],
        out_specs=pl.BlockSpec((2, f), lambda i: (0, 0)),
        out_shape=jax.ShapeDtypeStruct((2, f), jnp.float32),
        scratch_shapes=[pltpu.VMEM((2, f), jnp.float32)],
    )(x, g.reshape(1, f), b.reshape(1, f))


def _conv_tab_body(kvol, x_ref, ab_ref, w_ref, o_ref):
    a = ab_ref[0:1, :]
    b = ab_ref[1:2, :]
    h = x_ref[...] * a + b
    h = _silu(h)
    for k in range(kvol):
        m = jnp.dot(h, w_ref[k], preferred_element_type=jnp.float32)
        o_ref[0, k] = m[:, :128]
        o_ref[1, k] = m[:, 128:]


def _conv_table(x, ab, w, bn):
    n, f = x.shape
    kvol, _, fo = w.shape
    fh = fo // 2
    grid = n // bn
    tab = pl.pallas_call(
        functools.partial(_conv_tab_body, kvol),
        grid=(grid,),
        in_specs=[
            pl.BlockSpec((bn, f), lambda i: (i, 0)),
            pl.BlockSpec((2, f), lambda i: (0, 0)),
            pl.BlockSpec((kvol, f, fo), lambda i: (0, 0, 0)),
        ],
        out_specs=pl.BlockSpec((2, kvol, bn, fh), lambda i: (0, 0, i, 0)),
        out_shape=jax.ShapeDtypeStruct((2, kvol, n, fh), jnp.float32),
    )(x, ab, w)
    return tab.reshape(2 * kvol * n, fh)


def _sc_gs_body(nch, nacc, n_out, fh,
                tab, gidx4, dst3, zeros,
                out, idx_b, dst_b, rows0, rows1, acc, sem0, sem1):
    c = lax.axis_index("c")
    s = lax.axis_index("s")

    rz = nacc // _NSUB
    pltpu.sync_copy(zeros.at[pl.ds(s * rz, rz)], acc.at[pl.ds(s * rz, rz)])
    plsc.subcore_barrier()

    ngrp = nch // _GRP
    ghalf = _GRP // 2

    def group(g, carry):
        pltpu.sync_copy(gidx4.at[c, s, pl.ds(g * _GRP, _GRP)], idx_b)
        pltpu.sync_copy(dst3.at[s, pl.ds(g * _GRP, _GRP)], dst_b)
        pltpu.async_copy(tab.at[idx_b.at[0]], rows0, sem0)

        def body(i, carry2):
            j0 = 2 * i
            j1 = j0 + 1
            pltpu.make_async_copy(tab.at[idx_b.at[j0]], rows0, sem0).wait()
            pltpu.async_copy(tab.at[idx_b.at[j1]], rows1, sem1)
            pltpu.sync_copy(rows0, acc.at[dst_b.at[j0]], add=True)
            pltpu.make_async_copy(tab.at[idx_b.at[j1]], rows1, sem1).wait()

            @pl.when(i + 1 < ghalf)
            def _():
                pltpu.async_copy(tab.at[idx_b.at[j0 + 2]], rows0, sem0)

            pltpu.sync_copy(rows1, acc.at[dst_b.at[j1]], add=True)
            return carry2

        lax.fori_loop(0, ghalf, body, 0)
        return carry

    lax.fori_loop(0, ngrp, group, 0)
    plsc.subcore_barrier()

    ro = n_out // _NSUB
    pltpu.sync_copy(acc.at[pl.ds(s * ro, ro)],
                    out.at[c, pl.ds(s * ro, ro)])


def _sc_gather_scatter(tab, gidx4, dst3, zeros, n_out):
    fh = tab.shape[1]
    nch = dst3.shape[1]
    nacc = zeros.shape[0]
    mesh = plsc.VectorSubcoreMesh(core_axis_name="c", subcore_axis_name="s")
    body = functools.partial(_sc_gs_body, nch, nacc, n_out, fh)
    out = pl.kernel(
        body,
        out_type=jax.ShapeDtypeStruct((2, n_out, fh), jnp.float32),
        mesh=mesh,
        scratch_types=[
            pltpu.VMEM((_GRP, _CH), jnp.int32),
            pltpu.VMEM((_GRP, _CH), jnp.int32),
            pltpu.VMEM((_CH, fh), jnp.float32),
            pltpu.VMEM((_CH, fh), jnp.float32),
            pltpu.VMEM_SHARED((nacc, fh), jnp.float32),
            pltpu.SemaphoreType.DMA,
            pltpu.SemaphoreType.DMA,
        ],
    )(tab, gidx4, dst3, zeros)
    return out[0], out[1]


def _epi1_body(n_rows, nbatch, acca_ref, accb_ref, b1_ref, t_ref, wt_ref,
               bt_ref, bi_ref, h_ref, ab_ref, sacc_ref, temb_ref):
    i = pl.program_id(0)
    nb = pl.num_programs(0)

    @pl.when(i == 0)
    def _():
        temb_ref[...] = (
            jnp.dot(_silu(t_ref[...]), wt_ref[...],
                    preferred_element_type=jnp.float32) + bt_ref[...])
        sacc_ref[...] = jnp.zeros_like(sacc_ref)

    bi = bi_ref[0, 0, :]
    oh = (bi[:, None] == lax.broadcasted_iota(jnp.int32, (bi.shape[0], nbatch), 1)
          ).astype(jnp.float32)
    tembp = jnp.dot(oh, temb_ref[0:nbatch, :], preferred_element_type=jnp.float32)
    h = (jnp.concatenate([acca_ref[...], accb_ref[...]], axis=1)
         + b1_ref[...] + tembp)
    h_ref[...] = h

    sacc_ref[0:1, :] += jnp.sum(h, axis=0, keepdims=True)
    sacc_ref[1:2, :] += jnp.sum(h * h, axis=0, keepdims=True)

    @pl.when(i == nb - 1)
    def _():
        mu = sacc_ref[0:1, :] / n_rows
        var = sacc_ref[1:2, :] / n_rows - mu * mu
        r = lax.rsqrt(var + _EPS)
        ab_ref[0:1, :] = r
        ab_ref[1:2, :] = -mu * r


def _epilogue1(acca, accb, b1, t, wt, bt, bidx, g2, b2g, bn, n):
    fh = acca.shape[1]
    f = 2 * fh
    nbatch, temb_dim = t.shape
    tpad = jnp.zeros((8, temb_dim), jnp.float32).at[0:nbatch].set(t)
    bidx3 = bidx.reshape(n // bn, 1, bn)
    grid = n // bn
    h, ab = pl.pallas_call(
        functools.partial(_epi1_body, float(n), nbatch),
        grid=(grid,),
        in_specs=[
            pl.BlockSpec((bn, fh), lambda i: (i, 0)),
            pl.BlockSpec((bn, fh), lambda i: (i, 0)),
            pl.BlockSpec((1, f), lambda i: (0, 0)),
            pl.BlockSpec((8, temb_dim), lambda i: (0, 0)),
            pl.BlockSpec((temb_dim, f), lambda i: (0, 0)),
            pl.BlockSpec((1, f), lambda i: (0, 0)),
            pl.BlockSpec((1, 1, bn), lambda i: (i, 0, 0)),
        ],
        out_specs=[
            pl.BlockSpec((bn, f), lambda i: (i, 0)),
            pl.BlockSpec((2, f), lambda i: (0, 0)),
        ],
        out_shape=[
            jax.ShapeDtypeStruct((n, f), jnp.float32),
            jax.ShapeDtypeStruct((2, f), jnp.float32),
        ],
        scratch_shapes=[
            pltpu.VMEM((2, f), jnp.float32),
            pltpu.VMEM((8, f), jnp.float32),
        ],
    )(acca, accb, b1.reshape(1, f), tpad, wt, bt.reshape(1, f), bidx3)
    a = ab[0:1] * g2.reshape(1, f)
    b = ab[1:2] * g2.reshape(1, f) + b2g.reshape(1, f)
    return h, jnp.concatenate([a, b], axis=0)


def _final_body(acca_ref, accb_ref, b2_ref, x_ref, o_ref):
    o_ref[...] = (jnp.concatenate([acca_ref[...], accb_ref[...]], axis=1)
                  + b2_ref[...] + x_ref[...])


def _final(acca, accb, b2, x, bn):
    n, f = x.shape
    fh = acca.shape[1]
    grid = n // bn
    return pl.pallas_call(
        _final_body,
        grid=(grid,),
        in_specs=[
            pl.BlockSpec((bn, fh), lambda i: (i, 0)),
            pl.BlockSpec((bn, fh), lambda i: (i, 0)),
            pl.BlockSpec((1, f), lambda i: (0, 0)),
            pl.BlockSpec((bn, f), lambda i: (i, 0)),
        ],
        out_specs=pl.BlockSpec((bn, f), lambda i: (i, 0)),
        out_shape=jax.ShapeDtypeStruct((n, f), jnp.float32),
    )(acca, accb, b2.reshape(1, f), x)


def kernel(x, t, edge_index, kernel_offset, batch_idx,
           bn1_g, bn1_b, W1, b1, bn2_g, bn2_b, W2, b2, Wt, bt):
    n, f = x.shape
    e = edge_index.shape[1]
    kvol = W1.shape[0]
    bn = 200 if n % 200 == 0 else 8

    src = edge_index[0].astype(jnp.int32)
    dst = edge_index[1].astype(jnp.int32)
    koff = kernel_offset.astype(jnp.int32)
    gidx = koff * n + src
    bidx = batch_idx.astype(jnp.int32)

    step = _NSUB * _CH * _GRP
    e_pad = ((e + step - 1) // step) * step
    nch = e_pad // (_NSUB * _CH)
    n_out = ((n + 127) // 128) * 128
    if n_out > n:
        nacc, dump = n_out, n
    else:
        nacc, dump = n_out + 128, n_out
    gidx_pad = jnp.zeros((e_pad,), jnp.int32).at[:e].set(gidx)
    gidx4 = jnp.stack([gidx_pad, gidx_pad + kvol * n]).reshape(2, _NSUB, nch, _CH)
    dst3 = (jnp.full((e_pad,), dump, jnp.int32).at[:e].set(dst)
            .reshape(_NSUB, nch, _CH))
    zeros = jnp.zeros((nacc, f // 2), jnp.float32)

    ab1 = _bn_stats(x, bn1_g, bn1_b, bn)
    tab1 = _conv_table(x, ab1, W1, bn)
    acc1a, acc1b = _sc_gather_scatter(tab1, gidx4, dst3, zeros, n_out)

    h1, ab2 = _epilogue1(acc1a, acc1b, b1, t, Wt, bt, bidx, bn2_g, bn2_b, bn, n)

    tab2 = _conv_table(h1, ab2, W2, bn)
    acc2a, acc2b = _sc_gather_scatter(tab2, gidx4, dst3, zeros, n_out)

    return _final(acc2a, acc2b, b2, x, bn)


# speedup vs baseline: 1.9245x; 1.0056x over previous
import functools

import jax
import jax.numpy as jnp
from jax import lax
from jax.experimental import pallas as pl
from jax.experimental.pallas import tpu as pltpu
from jax.experimental.pallas import tpu_sc as plsc

_EPS = 1e-5
_CH = 128
_NSUB = 16
_GRP = 16


def _silu(v):
    return v * jax.nn.sigmoid(v)


def _stats_body(n_rows, x_ref, g_ref, b_ref, ab_ref, acc_ref):
    i = pl.program_id(0)
    nb = pl.num_programs(0)
    blk = x_ref[...]

    @pl.when(i == 0)
    def _():
        acc_ref[...] = jnp.zeros_like(acc_ref)

    acc_ref[0:1, :] += jnp.sum(blk, axis=0, keepdims=True)
    acc_ref[1:2, :] += jnp.sum(blk * blk, axis=0, keepdims=True)

    @pl.when(i == nb - 1)
    def _():
        mu = acc_ref[0:1, :] / n_rows
        var = acc_ref[1:2, :] / n_rows - mu * mu
        a = g_ref[...] * lax.rsqrt(var + _EPS)
        ab_ref[0:1, :] = a
        ab_ref[1:2, :] = b_ref[...] - mu * a


def _bn_stats(x, g, b, bn):
    n, f = x.shape
    grid = n // bn
    return pl.pallas_call(
        functools.partial(_stats_body, float(n)),
        grid=(grid,),
        in_specs=[
            pl.BlockSpec((bn, f), lambda i: (i, 0)),
            pl.BlockSpec((1, f), lambda i: (0, 0)),
            pl.BlockSpec((1, f), lambda i: (0, 0)),
        ],
        out_specs=pl.BlockSpec((2, f), lambda i: (0, 0)),
        out_shape=jax.ShapeDtypeStruct((2, f), jnp.float32),
        scratch_shapes=[pltpu.VMEM((2, f), jnp.float32)],
    )(x, g.reshape(1, f), b.reshape(1, f))


def _conv_tab_body(kvol, x_ref, ab_ref, w_ref, o_ref):
    a = ab_ref[0:1, :]
    b = ab_ref[1:2, :]
    h = x_ref[...] * a + b
    h = _silu(h).astype(jnp.bfloat16)
    for k in range(kvol):
        m = jnp.dot(h, w_ref[k], preferred_element_type=jnp.float32)
        o_ref[0, k] = m[:, :128]
        o_ref[1, k] = m[:, 128:]


def _conv_table(x, ab, w, bn):
    n, f = x.shape
    kvol, _, fo = w.shape
    fh = fo // 2
    grid = n // bn
    tab = pl.pallas_call(
        functools.partial(_conv_tab_body, kvol),
        grid=(grid,),
        in_specs=[
            pl.BlockSpec((bn, f), lambda i: (i, 0)),
            pl.BlockSpec((2, f), lambda i: (0, 0)),
            pl.BlockSpec((kvol, f, fo), lambda i: (0, 0, 0)),
        ],
        out_specs=pl.BlockSpec((2, kvol, bn, fh), lambda i: (0, 0, i, 0)),
        out_shape=jax.ShapeDtypeStruct((2, kvol, n, fh), jnp.float32),
    )(x, ab, w.astype(jnp.bfloat16))
    return tab.reshape(2 * kvol * n, fh)


def _sc_gs_body(nch, nacc, n_out, fh,
                tab, gidx4, dst3, zeros,
                out, idx_b, dst_b, rows0, rows1, acc, sem0, sem1):
    c = lax.axis_index("c")
    s = lax.axis_index("s")

    rz = nacc // _NSUB
    pltpu.sync_copy(zeros.at[pl.ds(s * rz, rz)], acc.at[pl.ds(s * rz, rz)])
    plsc.subcore_barrier()

    ngrp = nch // _GRP
    ghalf = _GRP // 2

    def group(g, carry):
        pltpu.sync_copy(gidx4.at[c, s, pl.ds(g * _GRP, _GRP)], idx_b)
        pltpu.sync_copy(dst3.at[s, pl.ds(g * _GRP, _GRP)], dst_b)
        pltpu.async_copy(tab.at[idx_b.at[0]], rows0, sem0)

        def body(i, carry2):
            j0 = 2 * i
            j1 = j0 + 1
            pltpu.make_async_copy(tab.at[idx_b.at[j0]], rows0, sem0).wait()
            pltpu.async_copy(tab.at[idx_b.at[j1]], rows1, sem1)
            pltpu.sync_copy(rows0, acc.at[dst_b.at[j0]], add=True)
            pltpu.make_async_copy(tab.at[idx_b.at[j1]], rows1, sem1).wait()

            @pl.when(i + 1 < ghalf)
            def _():
                pltpu.async_copy(tab.at[idx_b.at[j0 + 2]], rows0, sem0)

            pltpu.sync_copy(rows1, acc.at[dst_b.at[j1]], add=True)
            return carry2

        lax.fori_loop(0, ghalf, body, 0)
        return carry

    lax.fori_loop(0, ngrp, group, 0)
    plsc.subcore_barrier()

    ro = n_out // _NSUB
    pltpu.sync_copy(acc.at[pl.ds(s * ro, ro)],
                    out.at[c, pl.ds(s * ro, ro)])


def _sc_gather_scatter(tab, gidx4, dst3, zeros, n_out):
    fh = tab.shape[1]
    nch = dst3.shape[1]
    nacc = zeros.shape[0]
    mesh = plsc.VectorSubcoreMesh(core_axis_name="c", subcore_axis_name="s")
    body = functools.partial(_sc_gs_body, nch, nacc, n_out, fh)
    out = pl.kernel(
        body,
        out_type=jax.ShapeDtypeStruct((2, n_out, fh), jnp.float32),
        mesh=mesh,
        scratch_types=[
            pltpu.VMEM((_GRP, _CH), jnp.int32),
            pltpu.VMEM((_GRP, _CH), jnp.int32),
            pltpu.VMEM((_CH, fh), jnp.float32),
            pltpu.VMEM((_CH, fh), jnp.float32),
            pltpu.VMEM_SHARED((nacc, fh), jnp.float32),
            pltpu.SemaphoreType.DMA,
            pltpu.SemaphoreType.DMA,
        ],
    )(tab, gidx4, dst3, zeros)
    return out[0], out[1]


def _epi1_body(n_rows, nbatch, acca_ref, accb_ref, b1_ref, t_ref, wt_ref,
               bt_ref, bi_ref, h_ref, ab_ref, sacc_ref, temb_ref):
    i = pl.program_id(0)
    nb = pl.num_programs(0)

    @pl.when(i == 0)
    def _():
        temb_ref[...] = (
            jnp.dot(_silu(t_ref[...]), wt_ref[...],
                    preferred_element_type=jnp.float32) + bt_ref[...])
        sacc_ref[...] = jnp.zeros_like(sacc_ref)

    bi = bi_ref[0, 0, :]
    oh = (bi[:, None] == lax.broadcasted_iota(jnp.int32, (bi.shape[0], nbatch), 1)
          ).astype(jnp.float32)
    tembp = jnp.dot(oh, temb_ref[0:nbatch, :], preferred_element_type=jnp.float32)
    h = (jnp.concatenate([acca_ref[...], accb_ref[...]], axis=1)
         + b1_ref[...] + tembp)
    h_ref[...] = h

    sacc_ref[0:1, :] += jnp.sum(h, axis=0, keepdims=True)
    sacc_ref[1:2, :] += jnp.sum(h * h, axis=0, keepdims=True)

    @pl.when(i == nb - 1)
    def _():
        mu = sacc_ref[0:1, :] / n_rows
        var = sacc_ref[1:2, :] / n_rows - mu * mu
        r = lax.rsqrt(var + _EPS)
        ab_ref[0:1, :] = r
        ab_ref[1:2, :] = -mu * r


def _epilogue1(acca, accb, b1, t, wt, bt, bidx, g2, b2g, bn, n):
    fh = acca.shape[1]
    f = 2 * fh
    nbatch, temb_dim = t.shape
    tpad = jnp.zeros((8, temb_dim), jnp.float32).at[0:nbatch].set(t)
    bidx3 = bidx.reshape(n // bn, 1, bn)
    grid = n // bn
    h, ab = pl.pallas_call(
        functools.partial(_epi1_body, float(n), nbatch),
        grid=(grid,),
        in_specs=[
            pl.BlockSpec((bn, fh), lambda i: (i, 0)),
            pl.BlockSpec((bn, fh), lambda i: (i, 0)),
            pl.BlockSpec((1, f), lambda i: (0, 0)),
            pl.BlockSpec((8, temb_dim), lambda i: (0, 0)),
            pl.BlockSpec((temb_dim, f), lambda i: (0, 0)),
            pl.BlockSpec((1, f), lambda i: (0, 0)),
            pl.BlockSpec((1, 1, bn), lambda i: (i, 0, 0)),
        ],
        out_specs=[
            pl.BlockSpec((bn, f), lambda i: (i, 0)),
            pl.BlockSpec((2, f), lambda i: (0, 0)),
        ],
        out_shape=[
            jax.ShapeDtypeStruct((n, f), jnp.float32),
            jax.ShapeDtypeStruct((2, f), jnp.float32),
        ],
        scratch_shapes=[
            pltpu.VMEM((2, f), jnp.float32),
            pltpu.VMEM((8, f), jnp.float32),
        ],
    )(acca, accb, b1.reshape(1, f), tpad, wt, bt.reshape(1, f), bidx3)
    a = ab[0:1] * g2.reshape(1, f)
    b = ab[1:2] * g2.reshape(1, f) + b2g.reshape(1, f)
    return h, jnp.concatenate([a, b], axis=0)


def _final_body(acca_ref, accb_ref, b2_ref, x_ref, o_ref):
    o_ref[...] = (jnp.concatenate([acca_ref[...], accb_ref[...]], axis=1)
                  + b2_ref[...] + x_ref[...])


def _final(acca, accb, b2, x, bn):
    n, f = x.shape
    fh = acca.shape[1]
    grid = n // bn
    return pl.pallas_call(
        _final_body,
        grid=(grid,),
        in_specs=[
            pl.BlockSpec((bn, fh), lambda i: (i, 0)),
            pl.BlockSpec((bn, fh), lambda i: (i, 0)),
            pl.BlockSpec((1, f), lambda i: (0, 0)),
            pl.BlockSpec((bn, f), lambda i: (i, 0)),
        ],
        out_specs=pl.BlockSpec((bn, f), lambda i: (i, 0)),
        out_shape=jax.ShapeDtypeStruct((n, f), jnp.float32),
    )(acca, accb, b2.reshape(1, f), x)


def kernel(x, t, edge_index, kernel_offset, batch_idx,
           bn1_g, bn1_b, W1, b1, bn2_g, bn2_b, W2, b2, Wt, bt):
    n, f = x.shape
    e = edge_index.shape[1]
    kvol = W1.shape[0]
    bn = 200 if n % 200 == 0 else 8

    src = edge_index[0].astype(jnp.int32)
    dst = edge_index[1].astype(jnp.int32)
    koff = kernel_offset.astype(jnp.int32)
    gidx = koff * n + src
    bidx = batch_idx.astype(jnp.int32)

    step = _NSUB * _CH * _GRP
    e_pad = ((e + step - 1) // step) * step
    nch = e_pad // (_NSUB * _CH)
    n_out = ((n + 127) // 128) * 128
    if n_out > n:
        nacc, dump = n_out, n
    else:
        nacc, dump = n_out + 128, n_out
    gidx_pad = jnp.zeros((e_pad,), jnp.int32).at[:e].set(gidx)
    gidx4 = jnp.stack([gidx_pad, gidx_pad + kvol * n]).reshape(2, _NSUB, nch, _CH)
    dst3 = (jnp.full((e_pad,), dump, jnp.int32).at[:e].set(dst)
            .reshape(_NSUB, nch, _CH))
    zeros = jnp.zeros((nacc, f // 2), jnp.float32)

    ab1 = _bn_stats(x, bn1_g, bn1_b, bn)
    tab1 = _conv_table(x, ab1, W1, bn)
    acc1a, acc1b = _sc_gather_scatter(tab1, gidx4, dst3, zeros, n_out)

    h1, ab2 = _epilogue1(acc1a, acc1b, b1, t, Wt, bt, bidx, bn2_g, bn2_b, bn, n)

    tab2 = _conv_table(h1, ab2, W2, bn)
    acc2a, acc2b = _sc_gather_scatter(tab2, gidx4, dst3, zeros, n_out)

    return _final(acc2a, acc2b, b2, x, bn)
